# baseline (speedup 1.0000x reference)
import functools

import jax
import jax.numpy as jnp
from jax import lax
from jax.experimental import pallas as pl
from jax.experimental.pallas import tpu as pltpu
from jax.experimental.pallas import tpu_sc as plsc

VOCAB = 1000
D = 64
T = 200
B = 4096
N = B * T
NC = 2
NS = 16
NW = NC * NS
ROWS_PER_W = N // NW
CHUNK = 320
NCHUNKS = ROWS_PER_W // CHUNK
NBUF = 4
LOOK = 2


@functools.cache
def _build_sc_encode():
    mesh = plsc.VectorSubcoreMesh(core_axis_name="c", subcore_axis_name="s")
    return pl.kernel(
        _sc_encode_body,
        out_type=jax.ShapeDtypeStruct((N, D), jnp.float32),
        mesh=mesh,
        scratch_types=[
            pltpu.VMEM((ROWS_PER_W,), jnp.int32),
            [pltpu.VMEM((CHUNK, D), jnp.float32) for _ in range(NBUF)],
            pltpu.VMEM((T, D), jnp.float32),
            pltpu.VMEM_SHARED((VOCAB, D), jnp.float32),
            [pltpu.SemaphoreType.DMA for _ in range(NBUF)],
            [pltpu.SemaphoreType.DMA for _ in range(NBUF)],
        ],
        compiler_params=pltpu.CompilerParams(use_tc_tiling_on_sc=False),
    )


def _sc_encode_body(idx_hbm, table_hbm, pos_hbm, out_hbm,
                    idx_v, bufs, pos_v, table_sh, gsems, osems):
    sid = lax.axis_index("s")
    wid = sid * NC + lax.axis_index("c")
    base = wid * ROWS_PER_W

    @pl.when(sid == 0)
    def _():
        pltpu.sync_copy(table_hbm, table_sh)

    pltpu.sync_copy(
        idx_hbm.at[pl.ds(pl.multiple_of(wid * ROWS_PER_W, 8), ROWS_PER_W)], idx_v)
    pltpu.sync_copy(pos_hbm.at[pl.ds(0, T)], pos_v)
    plsc.subcore_barrier()

    def gather_descr(c, slot):
        return pltpu.make_async_copy(
            table_sh.at[idx_v.at[pl.ds(c * CHUNK, CHUNK)]],
            bufs[slot],
            gsems[slot],
        )

    def out_descr(c, slot):
        r0 = pl.multiple_of(base + c * CHUNK, 8)
        return pltpu.make_async_copy(
            bufs[slot],
            out_hbm.at[pl.ds(r0, CHUNK)],
            osems[slot],
        )

    for c in range(LOOK):
        gather_descr(c, c % NBUF).start()

    def ring_body(c4, _):
        for s in range(NBUF):
            c = NBUF * c4 + s
            sg = (s + LOOK) % NBUF

            @pl.when((c >= NBUF - LOOK) & (c + LOOK < NCHUNKS))
            def _():
                out_descr(c + LOOK - NBUF, sg).wait()

            @pl.when(c + LOOK < NCHUNKS)
            def _():
                gather_descr(c + LOOK, sg).start()

            gather_descr(c, s).wait()
            buf = bufs[s]

            def row_body(r, t):
                for j in range(D // 16):
                    plsc.addupdate(
                        buf.at[r, pl.ds(16 * j, 16)],
                        pos_v[t, pl.ds(16 * j, 16)],
                    )
                return lax.select(t == T - 1, 0, t + 1)

            lax.fori_loop(0, CHUNK, row_body, lax.rem(CHUNK * c, T), unroll=2)
            out_descr(c, s).start()
        return 0

    lax.fori_loop(0, NCHUNKS // NBUF, ring_body, 0)
    for c in range(NCHUNKS - NBUF, NCHUNKS):
        out_descr(c, c % NBUF).wait()


def kernel(raw_char_indices, emb0, emb1, emb2, emb3, pos_table):
    idx = raw_char_indices.astype(jnp.int32).reshape(N)
    table = jnp.concatenate([emb0, emb1, emb2, emb3], axis=1)
    out = _build_sc_encode()(idx, table, pos_table)
    return out.reshape(B, T, D)
